# baseline (speedup 1.0000x reference)
import functools

import jax
import jax.numpy as jnp
from jax import lax
from jax.experimental import pallas as pl
from jax.experimental.pallas import tpu as pltpu
from jax.experimental.pallas import tpu_sc as plsc

_E = 4
_D = 512
_H = 4 * _D
_N = 4096
_RT = 256
_NB = 6
_NT = _N // _RT + _NB
_NPAD = _NT * _RT
_NW = 32
_CHUNK = _N // _NW
_L = 16
_CW = 128


def _router_body(x_ref, wg_ref, c_ref, cpad_ref, gidw_ref, counts_ref,
                 xbf_ref):
    x = x_ref[...]
    xbf_ref[...] = x.astype(jnp.bfloat16)
    logits = jnp.dot(x, wg_ref[...], preferred_element_type=jnp.float32)
    m = jnp.max(logits, axis=-1, keepdims=True)
    ex = jnp.exp(logits - m)
    p = ex / jnp.sum(ex, axis=-1, keepdims=True)
    col = jax.lax.broadcasted_iota(jnp.int32, p.shape, 1)
    i1 = jnp.argmax(p, axis=-1)[:, None]
    m1 = jnp.max(p, axis=-1, keepdims=True)
    p_wo = jnp.where(col == i1, -jnp.inf, p)
    i2 = jnp.argmax(p_wo, axis=-1)[:, None]
    m2 = jnp.max(p_wo, axis=-1, keepdims=True)
    denom = jnp.maximum(m1 + m2, 1e-6)
    sel = (col == i1) | (col == i2)
    c = jnp.where(sel, p / denom, 0.0)
    c_ref[...] = c
    r = c.shape[0]
    cpad_ref[...] = jnp.concatenate(
        [c, jnp.zeros((r, _CW - _E), jnp.float32)], axis=1)
    a = jnp.minimum(i1, i2)
    b2 = jnp.maximum(i1, i2)
    gid = a * 3 - (a * (a - 1)) // 2 + (b2 - a - 1)
    col16 = jax.lax.broadcasted_iota(jnp.int32, (r, 16), 1)
    gidw_ref[...] = jnp.broadcast_to(gid, (r, 16))
    onehot = jnp.where(col16 == gid, 1, 0)
    counts_ref[...] = jnp.sum(
        onehot.reshape(r // _CHUNK, _CHUNK, 16), axis=1).astype(jnp.int32)


def _dispatch_body(x_hbm, gidw_hbm, cpad_hbm, counts_hbm,
                   xs_hbm, cg_hbm, pos_hbm, tmap_hbm,
                   gid_v, cnt_v, cpad_v, xrow_v, tmap_v, pos_v, sem):
    wid = lax.axis_index("s") * 2 + lax.axis_index("c")
    t0 = wid * _CHUNK
    pltpu.sync_copy(gidw_hbm.at[pl.ds(t0, _CHUNK)], gid_v)
    pltpu.sync_copy(counts_hbm, cnt_v)
    pltpu.sync_copy(cpad_hbm.at[pl.ds(t0, _CHUNK)], cpad_v)
    pltpu.sync_copy(x_hbm.at[pl.ds(t0, _CHUNK)], xrow_v)

    lanes = jax.lax.broadcasted_iota(jnp.int32, (_L,), 0)
    totals = jnp.zeros((_L,), jnp.int32)
    cnt_rows = []
    for w in range(_NW):
        row = cnt_v[w]
        cnt_rows.append(row)
        totals = totals + row
    padded = jnp.bitwise_and(totals + (_RT - 1), jnp.int32(~(_RT - 1)))

    base_s, incl_s = [], []
    run = jnp.int32(0)
    for bkt in range(_NB):
        base_s.append(run)
        run = run + padded[bkt]
        incl_s.append(run)
    sv = []
    for bkt in range(_NB):
        s = base_s[bkt]
        for w in range(_NW):
            s = s + jnp.where(w < wid, cnt_rows[w][bkt], 0)
        sv.append(jnp.full((_L,), s, jnp.int32))

    acc = jnp.zeros((_L,), jnp.int32)
    one = jnp.full((_L,), 1, jnp.int32)
    zero = jnp.zeros((_L,), jnp.int32)
    for t in range(_CHUNK):
        g = gid_v[t]
        k = t % _L
        slot = zero
        for bkt in range(_NB):
            m = g == bkt
            slot = slot + jnp.where(m, sv[bkt], zero)
            sv[bkt] = sv[bkt] + jnp.where(m, one, zero)
        acc = acc + jnp.where(lanes == k, slot, zero)
        if k == _L - 1:
            pos_v[pl.ds(t - _L + 1, _L)] = acc
            acc = zero

    pltpu.async_copy(xrow_v, xs_hbm.at[pos_v], sem).wait()
    pltpu.async_copy(cpad_v, cg_hbm.at[pos_v], sem).wait()
    pltpu.sync_copy(pos_v, pos_hbm.at[wid])

    @pl.when(wid == 0)
    def _():
        for j in range(2):
            tstart = (lanes + j * _L) * _RT
            tm = jnp.zeros((_L,), jnp.int32)
            for bkt in range(_NB):
                bv = jnp.full((_L,), incl_s[bkt], jnp.int32)
                tm = tm + jnp.where(bv <= tstart, one, zero)
            tm = jnp.minimum(tm, _NB - 1)
            tmap_v[pl.ds(j * _L, _L)] = tm
        pltpu.sync_copy(tmap_v, tmap_hbm)


def _moe_body(tmap_ref, xs_ref, w1_ref, w2_ref, cg_ref, ys_ref):
    i = pl.program_id(0)
    g = tmap_ref[i]
    a = jnp.where(g < 3, 0, jnp.where(g < 5, 1, 2))
    b = jnp.where(g < 3, g + 1, jnp.where(g < 5, g - 1, 3))
    x = xs_ref[...]
    cg = cg_ref[...]
    col = jax.lax.broadcasted_iota(jnp.int32, cg.shape, 1)
    wa = jnp.sum(jnp.where(col == a, cg, 0.0), axis=1, keepdims=True)
    wb = jnp.sum(jnp.where(col == b, cg, 0.0), axis=1, keepdims=True)
    acc = None
    for e, w in ((a, wa), (b, wb)):
        h = jnp.dot(x, w1_ref[e], preferred_element_type=jnp.float32)
        h = 0.5 * h * (1.0 + jax.lax.erf(h * 0.7071067811865476))
        y = jnp.dot(h.astype(jnp.bfloat16), w2_ref[e],
                    preferred_element_type=jnp.float32)
        acc = y * w if acc is None else acc + y * w
    ys_ref[...] = acc


def _combine_body(pos_hbm, ys_hbm, out_hbm, pos_v, rows_v, sem):
    wid = lax.axis_index("s") * 2 + lax.axis_index("c")
    pltpu.sync_copy(pos_hbm.at[wid], pos_v)
    pltpu.async_copy(ys_hbm.at[pos_v], rows_v, sem).wait()
    pltpu.sync_copy(rows_v, out_hbm.at[pl.ds(wid * _CHUNK, _CHUNK)])


def kernel(x, Wg, W1, W2):
    b, t, c = x.shape
    n = b * t
    x_flat = x.reshape(n, c)

    rb = 1024
    router = pl.pallas_call(
        _router_body,
        out_shape=(
            jax.ShapeDtypeStruct((n, _E), jnp.float32),
            jax.ShapeDtypeStruct((n, _CW), jnp.float32),
            jax.ShapeDtypeStruct((n, 16), jnp.int32),
            jax.ShapeDtypeStruct((_NW, 16), jnp.int32),
            jax.ShapeDtypeStruct((n, _D), jnp.bfloat16),
        ),
        grid=(n // rb,),
        in_specs=[
            pl.BlockSpec((rb, _D), lambda i: (i, 0)),
            pl.BlockSpec((_D, _E), lambda i: (0, 0)),
        ],
        out_specs=(
            pl.BlockSpec((rb, _E), lambda i: (i, 0)),
            pl.BlockSpec((rb, _CW), lambda i: (i, 0)),
            pl.BlockSpec((rb, 16), lambda i: (i, 0)),
            pl.BlockSpec((rb // _CHUNK, 16), lambda i: (i, 0)),
            pl.BlockSpec((rb, _D), lambda i: (i, 0)),
        ),
    )
    C, cpad, gidw, counts, xbf = router(x_flat, Wg)
    xi = jax.lax.bitcast_convert_type(
        xbf.reshape(n, _D // 2, 2), jnp.int32)

    mesh = plsc.VectorSubcoreMesh(core_axis_name="c", subcore_axis_name="s")
    dispatch = functools.partial(
        pl.kernel,
        out_type=(
            jax.ShapeDtypeStruct((_NPAD, _D // 2), jnp.int32),
            jax.ShapeDtypeStruct((_NPAD, _CW), jnp.float32),
            jax.ShapeDtypeStruct((_NW, _CHUNK), jnp.int32),
            jax.ShapeDtypeStruct((2 * _L,), jnp.int32),
        ),
        mesh=mesh,
        scratch_types=[
            pltpu.VMEM((_CHUNK, 16), jnp.int32),
            pltpu.VMEM((_NW, 16), jnp.int32),
            pltpu.VMEM((_CHUNK, _CW), jnp.float32),
            pltpu.VMEM((_CHUNK, _D // 2), jnp.int32),
            pltpu.VMEM((2 * _L,), jnp.int32),
            pltpu.VMEM((_CHUNK,), jnp.int32),
            pltpu.SemaphoreType.DMA,
        ],
    )(_dispatch_body)
    xsi, cg, pos, tmap = dispatch(xi, gidw, cpad, counts)
    xs = jax.lax.bitcast_convert_type(xsi, jnp.bfloat16).reshape(_NPAD, _D)

    moe = pl.pallas_call(
        _moe_body,
        out_shape=jax.ShapeDtypeStruct((_NPAD, _D), jnp.float32),
        grid_spec=pltpu.PrefetchScalarGridSpec(
            num_scalar_prefetch=1,
            grid=(_NT,),
            in_specs=[
                pl.BlockSpec((_RT, _D), lambda i, tm: (i, 0)),
                pl.BlockSpec((_E, _D, _H), lambda i, tm: (0, 0, 0)),
                pl.BlockSpec((_E, _H, _D), lambda i, tm: (0, 0, 0)),
                pl.BlockSpec((_RT, _CW), lambda i, tm: (i, 0)),
            ],
            out_specs=pl.BlockSpec((_RT, _D), lambda i, tm: (i, 0)),
        ),
    )
    ys = moe(tmap, xs, W1.astype(jnp.bfloat16), W2.astype(jnp.bfloat16), cg)

    combine = functools.partial(
        pl.kernel,
        out_type=jax.ShapeDtypeStruct((n, _D), jnp.float32),
        mesh=mesh,
        scratch_types=[
            pltpu.VMEM((_CHUNK,), jnp.int32),
            pltpu.VMEM((_CHUNK, _D), jnp.float32),
            pltpu.SemaphoreType.DMA,
        ],
    )(_combine_body)
    out_flat = combine(pos, ys)
    return out_flat.reshape(b, t, c), C


# speedup vs baseline: 2.0198x; 2.0198x over previous
import functools

import jax
import jax.numpy as jnp
from jax import lax
from jax.experimental import pallas as pl
from jax.experimental.pallas import tpu as pltpu
from jax.experimental.pallas import tpu_sc as plsc

_E = 4
_D = 512
_H = 4 * _D
_N = 4096
_RT = 256
_NB = 6
_NT = _N // _RT + _NB
_NPAD = _NT * _RT
_NW = 32
_CHUNK = _N // _NW
_L = 16
_CW = 128


def _router_body(x_ref, wg_ref, c_ref, cpad_ref, gidw_ref, counts_ref,
                 xi_ref):
    x = x_ref[...]
    hd = _D // 2
    lo = jax.lax.bitcast_convert_type(
        x[:, :hd].astype(jnp.bfloat16), jnp.uint16).astype(jnp.uint32)
    hi = jax.lax.bitcast_convert_type(
        x[:, hd:].astype(jnp.bfloat16), jnp.uint16).astype(jnp.uint32)
    xi_ref[...] = jax.lax.bitcast_convert_type(
        lo | (hi << 16), jnp.int32)
    logits = jnp.dot(x, wg_ref[...], preferred_element_type=jnp.float32)
    m = jnp.max(logits, axis=-1, keepdims=True)
    ex = jnp.exp(logits - m)
    p = ex / jnp.sum(ex, axis=-1, keepdims=True)
    col = jax.lax.broadcasted_iota(jnp.int32, p.shape, 1)
    i1 = jnp.argmax(p, axis=-1)[:, None]
    m1 = jnp.max(p, axis=-1, keepdims=True)
    p_wo = jnp.where(col == i1, -jnp.inf, p)
    i2 = jnp.argmax(p_wo, axis=-1)[:, None]
    m2 = jnp.max(p_wo, axis=-1, keepdims=True)
    denom = jnp.maximum(m1 + m2, 1e-6)
    sel = (col == i1) | (col == i2)
    c = jnp.where(sel, p / denom, 0.0)
    c_ref[...] = c
    r = c.shape[0]
    cpad_ref[...] = jnp.concatenate(
        [c, jnp.zeros((r, _CW - _E), jnp.float32)], axis=1)
    a = jnp.minimum(i1, i2)
    b2 = jnp.maximum(i1, i2)
    gid = a * 3 - (a * (a - 1)) // 2 + (b2 - a - 1)
    col16 = jax.lax.broadcasted_iota(jnp.int32, (r, 16), 1)
    gidw_ref[...] = jnp.broadcast_to(gid, (r, 16))
    onehot = jnp.where(col16 == gid, 1, 0)
    counts_ref[...] = jnp.sum(
        onehot.reshape(r // _CHUNK, _CHUNK, 16), axis=1).astype(jnp.int32)


def _dispatch_body(x_hbm, gidw_hbm, cpad_hbm, counts_hbm,
                   xs_hbm, cg_hbm, pos_hbm, tmap_hbm,
                   gid_v, cnt_v, cpad_v, xrow_v, tmap_v, pos_v, sem):
    wid = lax.axis_index("s") * 2 + lax.axis_index("c")
    t0 = wid * _CHUNK
    pltpu.sync_copy(gidw_hbm.at[pl.ds(t0, _CHUNK)], gid_v)
    pltpu.sync_copy(counts_hbm, cnt_v)
    pltpu.sync_copy(cpad_hbm.at[pl.ds(t0, _CHUNK)], cpad_v)
    pltpu.sync_copy(x_hbm.at[pl.ds(t0, _CHUNK)], xrow_v)

    lanes = jax.lax.broadcasted_iota(jnp.int32, (_L,), 0)
    totals = jnp.zeros((_L,), jnp.int32)
    cnt_rows = []
    for w in range(_NW):
        row = cnt_v[w]
        cnt_rows.append(row)
        totals = totals + row
    padded = jnp.bitwise_and(totals + (_RT - 1), jnp.int32(~(_RT - 1)))

    base_s, incl_s = [], []
    run = jnp.int32(0)
    for bkt in range(_NB):
        base_s.append(run)
        run = run + padded[bkt]
        incl_s.append(run)
    sv = []
    for bkt in range(_NB):
        s = base_s[bkt]
        for w in range(_NW):
            s = s + jnp.where(w < wid, cnt_rows[w][bkt], 0)
        sv.append(jnp.full((_L,), s, jnp.int32))

    acc = jnp.zeros((_L,), jnp.int32)
    one = jnp.full((_L,), 1, jnp.int32)
    zero = jnp.zeros((_L,), jnp.int32)
    for t in range(_CHUNK):
        g = gid_v[t]
        k = t % _L
        slot = zero
        for bkt in range(_NB):
            m = g == bkt
            slot = slot + jnp.where(m, sv[bkt], zero)
            sv[bkt] = sv[bkt] + jnp.where(m, one, zero)
        acc = acc + jnp.where(lanes == k, slot, zero)
        if k == _L - 1:
            pos_v[pl.ds(t - _L + 1, _L)] = acc
            acc = zero

    pltpu.async_copy(xrow_v, xs_hbm.at[pos_v], sem).wait()
    pltpu.async_copy(cpad_v, cg_hbm.at[pos_v], sem).wait()
    pltpu.sync_copy(pos_v, pos_hbm.at[wid])

    @pl.when(wid == 0)
    def _():
        for j in range(2):
            tstart = (lanes + j * _L) * _RT
            tm = jnp.zeros((_L,), jnp.int32)
            for bkt in range(_NB):
                bv = jnp.full((_L,), incl_s[bkt], jnp.int32)
                tm = tm + jnp.where(bv <= tstart, one, zero)
            tm = jnp.minimum(tm, _NB - 1)
            tmap_v[pl.ds(j * _L, _L)] = tm
        pltpu.sync_copy(tmap_v, tmap_hbm)


def _moe_body(tmap_ref, xs_ref, w1_ref, w2_ref, cg_ref, ys_ref):
    i = pl.program_id(0)
    g = tmap_ref[i]
    a = jnp.where(g < 3, 0, jnp.where(g < 5, 1, 2))
    b = jnp.where(g < 3, g + 1, jnp.where(g < 5, g - 1, 3))
    xi = jax.lax.bitcast_convert_type(xs_ref[...], jnp.uint32)
    lo = jax.lax.bitcast_convert_type(
        xi.astype(jnp.uint16), jnp.bfloat16)
    hi = jax.lax.bitcast_convert_type(
        (xi >> 16).astype(jnp.uint16), jnp.bfloat16)
    x = jnp.concatenate([lo, hi], axis=1)
    cg = cg_ref[...]
    col = jax.lax.broadcasted_iota(jnp.int32, cg.shape, 1)
    wa = jnp.sum(jnp.where(col == a, cg, 0.0), axis=1, keepdims=True)
    wb = jnp.sum(jnp.where(col == b, cg, 0.0), axis=1, keepdims=True)
    acc = None
    for e, w in ((a, wa), (b, wb)):
        h = jnp.dot(x, w1_ref[e], preferred_element_type=jnp.float32)
        h = 0.5 * h * (1.0 + jax.lax.erf(h * 0.7071067811865476))
        y = jnp.dot(h.astype(jnp.bfloat16), w2_ref[e],
                    preferred_element_type=jnp.float32)
        acc = y * w if acc is None else acc + y * w
    ys_ref[...] = acc


def _combine_body(pos_hbm, ys_hbm, out_hbm, pos_v, rows_v, sem):
    wid = lax.axis_index("s") * 2 + lax.axis_index("c")
    pltpu.sync_copy(pos_hbm.at[wid], pos_v)
    pltpu.async_copy(ys_hbm.at[pos_v], rows_v, sem).wait()
    pltpu.sync_copy(rows_v, out_hbm.at[pl.ds(wid * _CHUNK, _CHUNK)])


def kernel(x, Wg, W1, W2):
    b, t, c = x.shape
    n = b * t
    x_flat = x.reshape(n, c)

    rb = 1024
    router = pl.pallas_call(
        _router_body,
        out_shape=(
            jax.ShapeDtypeStruct((n, _E), jnp.float32),
            jax.ShapeDtypeStruct((n, _CW), jnp.float32),
            jax.ShapeDtypeStruct((n, 16), jnp.int32),
            jax.ShapeDtypeStruct((_NW, 16), jnp.int32),
            jax.ShapeDtypeStruct((n, _D // 2), jnp.int32),
        ),
        grid=(n // rb,),
        in_specs=[
            pl.BlockSpec((rb, _D), lambda i: (i, 0)),
            pl.BlockSpec((_D, _E), lambda i: (0, 0)),
        ],
        out_specs=(
            pl.BlockSpec((rb, _E), lambda i: (i, 0)),
            pl.BlockSpec((rb, _CW), lambda i: (i, 0)),
            pl.BlockSpec((rb, 16), lambda i: (i, 0)),
            pl.BlockSpec((rb // _CHUNK, 16), lambda i: (i, 0)),
            pl.BlockSpec((rb, _D // 2), lambda i: (i, 0)),
        ),
    )
    C, cpad, gidw, counts, xi = router(x_flat, Wg)

    mesh = plsc.VectorSubcoreMesh(core_axis_name="c", subcore_axis_name="s")
    dispatch = functools.partial(
        pl.kernel,
        out_type=(
            jax.ShapeDtypeStruct((_NPAD, _D // 2), jnp.int32),
            jax.ShapeDtypeStruct((_NPAD, _CW), jnp.float32),
            jax.ShapeDtypeStruct((_NW, _CHUNK), jnp.int32),
            jax.ShapeDtypeStruct((2 * _L,), jnp.int32),
        ),
        mesh=mesh,
        scratch_types=[
            pltpu.VMEM((_CHUNK, 16), jnp.int32),
            pltpu.VMEM((_NW, 16), jnp.int32),
            pltpu.VMEM((_CHUNK, _CW), jnp.float32),
            pltpu.VMEM((_CHUNK, _D // 2), jnp.int32),
            pltpu.VMEM((2 * _L,), jnp.int32),
            pltpu.VMEM((_CHUNK,), jnp.int32),
            pltpu.SemaphoreType.DMA,
        ],
    )(_dispatch_body)
    xs, cg, pos, tmap = dispatch(xi, gidw, cpad, counts)

    moe = pl.pallas_call(
        _moe_body,
        out_shape=jax.ShapeDtypeStruct((_NPAD, _D), jnp.float32),
        grid_spec=pltpu.PrefetchScalarGridSpec(
            num_scalar_prefetch=1,
            grid=(_NT,),
            in_specs=[
                pl.BlockSpec((_RT, _D // 2), lambda i, tm: (i, 0)),
                pl.BlockSpec((_E, _D, _H), lambda i, tm: (0, 0, 0)),
                pl.BlockSpec((_E, _H, _D), lambda i, tm: (0, 0, 0)),
                pl.BlockSpec((_RT, _CW), lambda i, tm: (i, 0)),
            ],
            out_specs=pl.BlockSpec((_RT, _D), lambda i, tm: (i, 0)),
        ),
    )
    ys = moe(tmap, xs, W1.astype(jnp.bfloat16), W2.astype(jnp.bfloat16), cg)

    combine = functools.partial(
        pl.kernel,
        out_type=jax.ShapeDtypeStruct((n, _D), jnp.float32),
        mesh=mesh,
        scratch_types=[
            pltpu.VMEM((_CHUNK,), jnp.int32),
            pltpu.VMEM((_CHUNK, _D), jnp.float32),
            pltpu.SemaphoreType.DMA,
        ],
    )(_combine_body)
    out_flat = combine(pos, ys)
    return out_flat.reshape(b, t, c), C
